# baseline (speedup 1.0000x reference)
import functools

import jax
import jax.numpy as jnp
from jax import lax
from jax.experimental import pallas as pl
from jax.experimental.pallas import tpu as pltpu
from jax.experimental.pallas import tpu_sc as plsc

DMODEL = 64
SCALE = 8.0

NC = 2
NS = 16
NW = NC * NS

CHUNK = 128


def _make_lookup(B, G):
    mesh = plsc.VectorSubcoreMesh(core_axis_name="c", subcore_axis_name="s")

    @functools.partial(
        pl.kernel,
        mesh=mesh,
        compiler_params=pltpu.CompilerParams(use_tc_tiling_on_sc=False),
        out_type=jax.ShapeDtypeStruct((B, DMODEL), jnp.float32),
        scratch_types=[
            pltpu.VMEM((G, CHUNK), jnp.int32),
            pltpu.VMEM((CHUNK, DMODEL), jnp.float32),
            pltpu.SemaphoreType.DMA,
            pltpu.SemaphoreType.DMA,
        ],
    )
    def lookup(idx_hbm, table_hbm, out_hbm, idx_v, buf, gsem, ssem):
        wid = lax.axis_index("s") * NC + lax.axis_index("c")
        pltpu.sync_copy(idx_hbm.at[wid], idx_v)
        base = wid * (G * CHUNK)

        def chunk_body(g, carry):
            pltpu.async_copy(table_hbm.at[idx_v.at[g]], buf, gsem).wait()

            def scale_row(r, c):
                for k in range(DMODEL // 16):
                    buf[r, pl.ds(k * 16, 16)] = buf[r, pl.ds(k * 16, 16)] * SCALE
                return c

            lax.fori_loop(0, CHUNK, scale_row, 0)
            pltpu.async_copy(buf, out_hbm.at[pl.ds(base + g * CHUNK, CHUNK)], ssem).wait()
            return carry

        lax.fori_loop(0, G, chunk_body, 0)

    return lookup


def kernel(input_sentence, table):
    S, T = input_sentence.shape
    B = S * T
    per_w = B // NW
    G = per_w // CHUNK
    idx = input_sentence.reshape(NW, G, CHUNK).astype(jnp.int32)
    out = _make_lookup(B, G)(idx, table)
    return out.reshape(S, T, DMODEL)


# speedup vs baseline: 1.2082x; 1.2082x over previous
import functools

import jax
import jax.numpy as jnp
from jax import lax
from jax.experimental import pallas as pl
from jax.experimental.pallas import tpu as pltpu
from jax.experimental.pallas import tpu_sc as plsc

DMODEL = 64
SCALE = 8.0

NC = 2
NS = 16
NW = NC * NS

CHUNK = 128
SB = 2
RING = 4


def _make_lookup(nrows, nsb):
    mesh = plsc.VectorSubcoreMesh(core_axis_name="c", subcore_axis_name="s")
    buf_t = pltpu.VMEM((SB, CHUNK, DMODEL), jnp.float32)

    @functools.partial(
        pl.kernel,
        mesh=mesh,
        compiler_params=pltpu.CompilerParams(use_tc_tiling_on_sc=False),
        out_type=jax.ShapeDtypeStruct((nrows, CHUNK, DMODEL), jnp.float32),
        scratch_types=[
            pltpu.VMEM((nsb, SB, CHUNK), jnp.int32),
            buf_t, buf_t, buf_t, buf_t,
            pltpu.SemaphoreType.DMA, pltpu.SemaphoreType.DMA,
            pltpu.SemaphoreType.DMA, pltpu.SemaphoreType.DMA,
            pltpu.SemaphoreType.DMA, pltpu.SemaphoreType.DMA,
            pltpu.SemaphoreType.DMA, pltpu.SemaphoreType.DMA,
        ],
    )
    def lookup(idx_hbm, table_hbm, out_hbm, idx_v,
               buf0, buf1, buf2, buf3,
               g0, g1, g2, g3, s0, s1, s2, s3):
        bufs = (buf0, buf1, buf2, buf3)
        gsems = (g0, g1, g2, g3)
        ssems = (s0, s1, s2, s3)
        wid = lax.axis_index("s") * NC + lax.axis_index("c")
        pltpu.sync_copy(idx_hbm.at[wid], idx_v)
        base = wid * (nsb * SB)

        def fire_gather(sb, b):
            for k in range(SB):
                pltpu.make_async_copy(
                    table_hbm.at[idx_v.at[sb, k]], bufs[b].at[k], gsems[b]).start()

        def wait_gather(sb, b):
            for k in range(SB):
                pltpu.make_async_copy(
                    table_hbm.at[idx_v.at[sb, k]], bufs[b].at[k], gsems[b]).wait()

        def fire_scatter(sb, b):
            pltpu.make_async_copy(
                bufs[b], out_hbm.at[pl.ds(base + sb * SB, SB)], ssems[b]).start()

        def wait_scatter(sb, b):
            pltpu.make_async_copy(
                bufs[b], out_hbm.at[pl.ds(base + sb * SB, SB)], ssems[b]).wait()

        def scale(b):
            buf = bufs[b]
            for k2 in range(SB):
                def scale_rows(i, c, _k2=k2):
                    r = i * 4
                    for rr in range(4):
                        for k in range(DMODEL // 16):
                            sl = pl.ds(k * 16, 16)
                            buf[_k2, r + rr, sl] = buf[_k2, r + rr, sl] * SCALE
                    return c
                lax.fori_loop(0, CHUNK // 4, scale_rows, 0)

        fire_gather(0, 0)

        def outer(s, carry):
            for j in range(RING):
                sb = s * RING + j
                jn = (j + 1) % RING

                @pl.when(sb >= RING - 1)
                def _():
                    wait_scatter(sb - (RING - 1), jn)

                @pl.when(sb + 1 < nsb)
                def _():
                    fire_gather(sb + 1, jn)

                wait_gather(sb, j)
                scale(j)
                fire_scatter(sb, j)
            return carry

        lax.fori_loop(0, nsb // RING, outer, 0)

        for sb in range(nsb - (RING - 1), nsb):
            wait_scatter(sb, sb % RING)

    return lookup


def kernel(input_sentence, table):
    S, T = input_sentence.shape
    B = S * T
    nrows = B // CHUNK
    nsb = B // (NW * SB * CHUNK)
    idx = input_sentence.reshape(NW, nsb, SB, CHUNK).astype(jnp.int32)
    out = _make_lookup(nrows, nsb)(idx, table)
    return out.reshape(S, T, DMODEL)
